# baseline (speedup 1.0000x reference)
import functools

import jax
import jax.numpy as jnp
from jax.experimental import pallas as pl
from jax.experimental.pallas import tpu as pltpu
from jax.experimental.pallas import tpu_sc as plsc

KNN = 20
NPTS = 10000
_ROWS_A = 400
_ROWS_B = 1000
_NPAD = 10112
_GPAD = 10240


_NW = 32


def _sc_gather(table, flat_idx, nchunks):
    num = flat_idx.shape[0]
    dim = table.shape[1]
    bpw = num // _NW
    chunk = bpw // nchunks
    mesh = plsc.VectorSubcoreMesh(core_axis_name="c", subcore_axis_name="s")

    @functools.partial(
        pl.kernel,
        out_type=jax.ShapeDtypeStruct((num, dim), table.dtype),
        mesh=mesh,
        scratch_types=[
            pltpu.VMEM((chunk,), jnp.int32),
            pltpu.VMEM((chunk, dim), jnp.float32),
            pltpu.SemaphoreType.DMA,
        ],
    )
    def gk(table_hbm, idx_hbm, out_hbm, idx_v, rows_v, sem):
        wid = jax.lax.axis_index("s") * 2 + jax.lax.axis_index("c")
        base = wid * bpw

        @pl.loop(0, nchunks)
        def _(c):
            off = base + c * chunk
            pltpu.sync_copy(idx_hbm.at[pl.ds(off, chunk)], idx_v)
            pltpu.async_copy(table_hbm.at[idx_v], rows_v, sem).wait()
            pltpu.sync_copy(rows_v, out_hbm.at[pl.ds(off, chunk)])

    return gk(table, flat_idx)


def _knn_kernel(x_ref, xt_ref, idx_ref):
    xb = x_ref[...]
    xt = xt_ref[...]
    sqc = jnp.sum(xt * xt, axis=0, keepdims=True)
    sqr = jnp.sum(xb * xb, axis=1, keepdims=True)
    dot = jnp.dot(xb, xt, preferred_element_type=jnp.float32)
    dist = (sqr + sqc) - 2.0 * dot
    r = dist.shape[0]
    ntiles = _NPAD // 128

    big = jnp.float32(3.4e38)
    depth = 4
    ms = [jnp.full((r, 128), big) for _ in range(depth)]
    gs = [jnp.zeros((r, 128), jnp.int32) for _ in range(depth)]
    for t in range(ntiles):
        cv = dist[:, t * 128:(t + 1) * 128]
        cg = jnp.full((r, 128), t, jnp.int32)
        for lvl in range(depth):
            c = cv < ms[lvl]
            if lvl < depth - 1:
                ms[lvl], cv = jnp.where(c, cv, ms[lvl]), jnp.where(c, ms[lvl], cv)
                gs[lvl], cg = jnp.where(c, cg, gs[lvl]), jnp.where(c, gs[lvl], cg)
            else:
                ms[lvl] = jnp.where(c, cv, ms[lvl])
                gs[lvl] = jnp.where(c, cg, gs[lvl])

    lane = jax.lax.broadcasted_iota(jnp.int32, (r, 128), 1)
    pool = jnp.concatenate(ms, axis=1)
    pool_j = jnp.concatenate([g * 128 + lane for g in gs], axis=1)
    for k in range(KNN):
        m = jnp.min(pool, axis=1, keepdims=True)
        j = jnp.min(jnp.where(pool == m, pool_j, NPTS * 2), axis=1,
                    keepdims=True)
        idx_ref[:, k] = j[:, 0]
        pool = jnp.where((pool == m) & (pool_j == j), big, pool)


def _edge_kernel(fdim, x_ref, xg_ref, w1_ref, a1_ref, w2_ref, a2_ref, o_ref):
    xb = x_ref[...]
    w1 = w1_ref[...]
    w2 = w2_ref[...]
    c1 = a1_ref[0:1]
    s1 = a1_ref[1:2]
    t1 = a1_ref[2:3]
    c2 = a2_ref[0:1]
    s2 = a2_ref[1:2]
    t2 = a2_ref[2:3]
    acc = None
    for k in range(KNN):
        msg = jnp.concatenate([xb, xg_ref[k][:, :fdim] - xb], axis=1)
        h = jnp.dot(msg, w1, preferred_element_type=jnp.float32) + c1
        h = jnp.maximum(h * s1 + t1, 0.0)
        e = jnp.dot(h, w2, preferred_element_type=jnp.float32) + c2
        e = e * s2 + t2
        acc = e if acc is None else jnp.maximum(acc, e)
    o_ref[...] = jnp.maximum(acc, 0.0)


def _cls_kernel(f_ref, w1_ref, a1_ref, w2_ref, b2_ref, o_ref):
    c1 = a1_ref[0:1]
    s1 = a1_ref[1:2]
    t1 = a1_ref[2:3]
    h = jnp.dot(f_ref[...], w1_ref[...], preferred_element_type=jnp.float32) + c1
    h = jnp.maximum(h * s1 + t1, 0.0)
    o_ref[...] = jnp.dot(h, w2_ref[...], preferred_element_type=jnp.float32) + b2_ref[...]


def _bn_scale_shift(g, b, rm, rv):
    s = g / jnp.sqrt(rv + 1e-5)
    return s, b - rm * s


def _edge_conv(x, p, fdim, hdim):
    n = x.shape[0]
    grid_a = pl.cdiv(n, _ROWS_A)
    xtp = jnp.pad(x.T, ((0, 0), (0, _NPAD - n)), constant_values=1e6)
    idx = pl.pallas_call(
        _knn_kernel,
        grid=(grid_a,),
        in_specs=[
            pl.BlockSpec((_ROWS_A, fdim), lambda i: (i, 0)),
            pl.BlockSpec((fdim, _NPAD), lambda i: (0, 0)),
        ],
        out_specs=pl.BlockSpec((_ROWS_A, KNN), lambda i: (i, 0)),
        out_shape=jax.ShapeDtypeStruct((n, KNN), jnp.int32),
    )(x, xtp)

    flat_idx = jnp.pad(idx.T, ((0, 0), (0, _GPAD - n))).reshape(-1)
    gdim = 128
    table = jnp.pad(x, ((0, 0), (0, gdim - fdim)))
    xg = _sc_gather(table, flat_idx, nchunks=8).reshape(KNN, _GPAD, gdim)

    s1, t1 = _bn_scale_shift(p['g1'], p['be1'], p['rm1'], p['rv1'])
    s2, t2 = _bn_scale_shift(p['g2'], p['be2'], p['rm2'], p['rv2'])
    a1 = jnp.stack([p['c1'], s1, t1])
    a2 = jnp.stack([p['c2'], s2, t2])

    grid_b = pl.cdiv(n, _ROWS_B)
    out = pl.pallas_call(
        functools.partial(_edge_kernel, fdim),
        grid=(grid_b,),
        in_specs=[
            pl.BlockSpec((_ROWS_B, fdim), lambda i: (i, 0)),
            pl.BlockSpec((KNN, _ROWS_B, 128), lambda i: (0, i, 0)),
            pl.BlockSpec((2 * fdim, hdim), lambda i: (0, 0)),
            pl.BlockSpec((3, hdim), lambda i: (0, 0)),
            pl.BlockSpec((hdim, hdim), lambda i: (0, 0)),
            pl.BlockSpec((3, hdim), lambda i: (0, 0)),
        ],
        out_specs=pl.BlockSpec((_ROWS_B, hdim), lambda i: (i, 0)),
        out_shape=jax.ShapeDtypeStruct((n, hdim), jnp.float32),
    )(x, xg, p['W1'], a1, p['W2'], a2)
    return out


def _bn_in(x, p):
    return (x - p['rm']) / jnp.sqrt(p['rv'] + 1e-5) * p['g'] + p['b']


def kernel(x, batch, params):
    p = params
    xs = _bn_in(x[:, :3], p['bn_spatial'])
    xt = _bn_in(x[:, 3:4], p['bn_time'])
    xp = jnp.concatenate([xs, xt], axis=1)

    out_s = _edge_conv(xs, p['conv_spatial'], 3, 32)
    out_p = _edge_conv(xp, p['conv_persistence'], 4, 32)
    comb = jnp.concatenate([out_s, out_p], axis=1)
    out_f = _edge_conv(comb, p['conv_fusion'], 64, 64)
    final = jnp.concatenate([out_s, out_p, out_f], axis=1)

    s1, t1 = _bn_scale_shift(p['cls_bn']['g'], p['cls_bn']['b'],
                             p['cls_bn']['rm'], p['cls_bn']['rv'])
    a1 = jnp.stack([p['cls_c1'], s1, t1])

    n = final.shape[0]
    grid_c = pl.cdiv(n, _ROWS_B)
    out = pl.pallas_call(
        _cls_kernel,
        grid=(grid_c,),
        in_specs=[
            pl.BlockSpec((_ROWS_B, 128), lambda i: (i, 0)),
            pl.BlockSpec((128, 128), lambda i: (0, 0)),
            pl.BlockSpec((3, 128), lambda i: (0, 0)),
            pl.BlockSpec((128, 1), lambda i: (0, 0)),
            pl.BlockSpec((1, 1), lambda i: (0, 0)),
        ],
        out_specs=pl.BlockSpec((_ROWS_B, 1), lambda i: (i, 0)),
        out_shape=jax.ShapeDtypeStruct((n, 1), jnp.float32),
    )(final, p['cls_W1'], a1, p['cls_W2'], p['cls_c2'][None, :])
    return out


# speedup vs baseline: 1.1219x; 1.1219x over previous
import functools

import jax
import jax.numpy as jnp
from jax.experimental import pallas as pl
from jax.experimental.pallas import tpu as pltpu
from jax.experimental.pallas import tpu_sc as plsc

KNN = 20
NPTS = 10000
_ROWS_A = 400
_ROWS_B = 1000
_NPAD = 10112


_NW = 32


def _sc_gather(table, flat_idx, nchunks):
    num = flat_idx.shape[0]
    dim = table.shape[1]
    bpw = num // _NW
    chunk = bpw // nchunks
    mesh = plsc.VectorSubcoreMesh(core_axis_name="c", subcore_axis_name="s")

    @functools.partial(
        pl.kernel,
        out_type=jax.ShapeDtypeStruct((num, dim), table.dtype),
        mesh=mesh,
        scratch_types=[
            pltpu.VMEM((chunk,), jnp.int32),
            pltpu.VMEM((chunk, dim), jnp.float32),
            pltpu.SemaphoreType.DMA,
        ],
    )
    def gk(table_hbm, idx_hbm, out_hbm, idx_v, rows_v, sem):
        wid = jax.lax.axis_index("s") * 2 + jax.lax.axis_index("c")
        base = wid * bpw

        @pl.loop(0, nchunks)
        def _(c):
            off = base + c * chunk
            pltpu.sync_copy(idx_hbm.at[pl.ds(off, chunk)], idx_v)
            pltpu.async_copy(table_hbm.at[idx_v], rows_v, sem).wait()
            pltpu.sync_copy(rows_v, out_hbm.at[pl.ds(off, chunk)])

    return gk(table, flat_idx)


def _knn_kernel(x_ref, xt_ref, idx_ref):
    xb = x_ref[...]
    xt = xt_ref[...]
    sqc = jnp.sum(xt * xt, axis=0, keepdims=True)
    sqr = jnp.sum(xb * xb, axis=1, keepdims=True)
    dot = jnp.dot(xb, xt, preferred_element_type=jnp.float32)
    dist = (sqr + sqc) - 2.0 * dot
    r = dist.shape[0]
    ntiles = _NPAD // 128

    big = jnp.float32(3.4e38)
    depth = 4
    ms = [jnp.full((r, 128), big) for _ in range(depth)]
    gs = [jnp.zeros((r, 128), jnp.int32) for _ in range(depth)]
    for t in range(ntiles):
        cv = dist[:, t * 128:(t + 1) * 128]
        cg = jnp.full((r, 128), t, jnp.int32)
        for lvl in range(depth):
            c = cv < ms[lvl]
            if lvl < depth - 1:
                ms[lvl], cv = jnp.where(c, cv, ms[lvl]), jnp.where(c, ms[lvl], cv)
                gs[lvl], cg = jnp.where(c, cg, gs[lvl]), jnp.where(c, gs[lvl], cg)
            else:
                ms[lvl] = jnp.where(c, cv, ms[lvl])
                gs[lvl] = jnp.where(c, cg, gs[lvl])

    lane = jax.lax.broadcasted_iota(jnp.int32, (r, 128), 1)
    pool = jnp.concatenate(ms, axis=1)
    pool_j = jnp.concatenate([g * 128 + lane for g in gs], axis=1)
    for k in range(KNN):
        m = jnp.min(pool, axis=1, keepdims=True)
        j = jnp.min(jnp.where(pool == m, pool_j, NPTS * 2), axis=1,
                    keepdims=True)
        idx_ref[:, k] = j[:, 0]
        pool = jnp.where((pool == m) & (pool_j == j), big, pool)


def _edge_kernel(fdim, x_ref, xg_ref, w1_ref, a1_ref, w2_ref, a2_ref, o_ref):
    xb = x_ref[...]
    w1 = w1_ref[...]
    w2 = w2_ref[...]
    c1 = a1_ref[0:1]
    s1 = a1_ref[1:2]
    t1 = a1_ref[2:3]
    c2 = a2_ref[0:1]
    s2 = a2_ref[1:2]
    t2 = a2_ref[2:3]
    acc = None
    for k in range(KNN):
        msg = jnp.concatenate([xb, xg_ref[k][:, :fdim] - xb], axis=1)
        h = jnp.dot(msg, w1, preferred_element_type=jnp.float32) + c1
        h = jnp.maximum(h * s1 + t1, 0.0)
        e = jnp.dot(h, w2, preferred_element_type=jnp.float32) + c2
        e = e * s2 + t2
        acc = e if acc is None else jnp.maximum(acc, e)
    o_ref[...] = jnp.maximum(acc, 0.0)


def _cls_kernel(f_ref, w1_ref, a1_ref, w2_ref, b2_ref, o_ref):
    c1 = a1_ref[0:1]
    s1 = a1_ref[1:2]
    t1 = a1_ref[2:3]
    h = jnp.dot(f_ref[...], w1_ref[...], preferred_element_type=jnp.float32) + c1
    h = jnp.maximum(h * s1 + t1, 0.0)
    o_ref[...] = jnp.dot(h, w2_ref[...], preferred_element_type=jnp.float32) + b2_ref[...]


def _bn_scale_shift(g, b, rm, rv):
    s = g / jnp.sqrt(rv + 1e-5)
    return s, b - rm * s


def _edge_conv(x, p, fdim, hdim):
    n = x.shape[0]
    grid_a = pl.cdiv(n, _ROWS_A)
    xtp = jnp.pad(x.T, ((0, 0), (0, _NPAD - n)), constant_values=1e6)
    idx = pl.pallas_call(
        _knn_kernel,
        grid=(grid_a,),
        in_specs=[
            pl.BlockSpec((_ROWS_A, fdim), lambda i: (i, 0)),
            pl.BlockSpec((fdim, _NPAD), lambda i: (0, 0)),
        ],
        out_specs=pl.BlockSpec((_ROWS_A, KNN), lambda i: (i, 0)),
        out_shape=jax.ShapeDtypeStruct((n, KNN), jnp.int32),
    )(x, xtp)

    flat_idx = jnp.pad(idx.T, ((0, 0), (0, _NPAD - n))).reshape(-1)
    gdim = 128
    table = jnp.pad(x, ((0, 0), (0, gdim - fdim)))
    xg = _sc_gather(table, flat_idx, nchunks=10).reshape(KNN, _NPAD, gdim)

    s1, t1 = _bn_scale_shift(p['g1'], p['be1'], p['rm1'], p['rv1'])
    s2, t2 = _bn_scale_shift(p['g2'], p['be2'], p['rm2'], p['rv2'])
    a1 = jnp.stack([p['c1'], s1, t1])
    a2 = jnp.stack([p['c2'], s2, t2])

    grid_b = pl.cdiv(n, _ROWS_B)
    out = pl.pallas_call(
        functools.partial(_edge_kernel, fdim),
        grid=(grid_b,),
        in_specs=[
            pl.BlockSpec((_ROWS_B, fdim), lambda i: (i, 0)),
            pl.BlockSpec((KNN, _ROWS_B, 128), lambda i: (0, i, 0)),
            pl.BlockSpec((2 * fdim, hdim), lambda i: (0, 0)),
            pl.BlockSpec((3, hdim), lambda i: (0, 0)),
            pl.BlockSpec((hdim, hdim), lambda i: (0, 0)),
            pl.BlockSpec((3, hdim), lambda i: (0, 0)),
        ],
        out_specs=pl.BlockSpec((_ROWS_B, hdim), lambda i: (i, 0)),
        out_shape=jax.ShapeDtypeStruct((n, hdim), jnp.float32),
    )(x, xg, p['W1'], a1, p['W2'], a2)
    return out


def _bn_in(x, p):
    return (x - p['rm']) / jnp.sqrt(p['rv'] + 1e-5) * p['g'] + p['b']


def kernel(x, batch, params):
    p = params
    xs = _bn_in(x[:, :3], p['bn_spatial'])
    xt = _bn_in(x[:, 3:4], p['bn_time'])
    xp = jnp.concatenate([xs, xt], axis=1)

    out_s = _edge_conv(xs, p['conv_spatial'], 3, 32)
    out_p = _edge_conv(xp, p['conv_persistence'], 4, 32)
    comb = jnp.concatenate([out_s, out_p], axis=1)
    out_f = _edge_conv(comb, p['conv_fusion'], 64, 64)
    final = jnp.concatenate([out_s, out_p, out_f], axis=1)

    s1, t1 = _bn_scale_shift(p['cls_bn']['g'], p['cls_bn']['b'],
                             p['cls_bn']['rm'], p['cls_bn']['rv'])
    a1 = jnp.stack([p['cls_c1'], s1, t1])

    n = final.shape[0]
    grid_c = pl.cdiv(n, _ROWS_B)
    out = pl.pallas_call(
        _cls_kernel,
        grid=(grid_c,),
        in_specs=[
            pl.BlockSpec((_ROWS_B, 128), lambda i: (i, 0)),
            pl.BlockSpec((128, 128), lambda i: (0, 0)),
            pl.BlockSpec((3, 128), lambda i: (0, 0)),
            pl.BlockSpec((128, 1), lambda i: (0, 0)),
            pl.BlockSpec((1, 1), lambda i: (0, 0)),
        ],
        out_specs=pl.BlockSpec((_ROWS_B, 1), lambda i: (i, 0)),
        out_shape=jax.ShapeDtypeStruct((n, 1), jnp.float32),
    )(final, p['cls_W1'], a1, p['cls_W2'], p['cls_c2'][None, :])
    return out
